# baseline (speedup 1.0000x reference)
import functools

import jax
import jax.numpy as jnp
from jax import lax
from jax.experimental import pallas as pl
from jax.experimental.pallas import tpu as pltpu
from jax.experimental.pallas import tpu_sc as plsc

_VOCAB = 8192
_DIM = 256
_BETA = 0.25
_B = 8
_HW = 1024
_N = _B * _HW
_NB = 1024
_KB = 1024
_NSTEP_K = _VOCAB // _KB


def _argmin_kernel(z_ref, e_ref, idx_ref, loss_ref,
                   zrows_s, x2_s, bv_s, bi_s, loss_s):
    k = pl.program_id(1)
    n = pl.program_id(0)

    @pl.when(k == 0)
    def _prep():
        z_rows = jnp.transpose(z_ref[0], (1, 0))
        zrows_s[...] = z_rows
        x2_s[...] = jnp.sum(z_rows * z_rows, axis=1, keepdims=True)

    z_rows = zrows_s[...]
    e = e_ref[...]
    y2 = jnp.sum(e * e, axis=1, keepdims=True)
    mm = lax.dot_general(z_rows, e, (((1,), (1,)), ((), ())),
                         precision=lax.Precision.DEFAULT,
                         preferred_element_type=jnp.float32)
    d2 = (x2_s[...] + jnp.transpose(y2, (1, 0))) - 2.0 * mm
    dist = jnp.sqrt(jnp.maximum(d2, 0.0))
    bv = jnp.min(dist, axis=1, keepdims=True)
    ii = lax.broadcasted_iota(jnp.int32, dist.shape, 1) + k * _KB
    bi = jnp.min(jnp.where(dist == bv, ii, jnp.int32(2 ** 30)),
                 axis=1, keepdims=True)

    @pl.when(k == 0)
    def _init():
        bv_s[...] = bv
        bi_s[...] = bi

    @pl.when(k > 0)
    def _merge():
        old_v = bv_s[...]
        upd = bv < old_v
        bv_s[...] = jnp.where(upd, bv, old_v)
        bi_s[...] = jnp.where(upd, bi, bi_s[...])

    @pl.when(k == _NSTEP_K - 1)
    def _fin():
        idx_ref[0, 0, :] = jnp.reshape(jnp.transpose(bi_s[...], (1, 0)),
                                       (1, _NB))[0]
        v = bv_s[...]
        part = jnp.sum(v * v)

        @pl.when(n == 0)
        def _():
            loss_s[0, 0] = part

        @pl.when(n > 0)
        def _():
            loss_s[0, 0] = loss_s[0, 0] + part

        @pl.when(n == _B - 1)
        def _():
            loss_ref[0, 0] = loss_s[0, 0] * (_BETA / (_N * _DIM))


def _distance_argmin(z_cn, emb):
    return pl.pallas_call(
        _argmin_kernel,
        grid=(_B, _NSTEP_K),
        in_specs=[
            pl.BlockSpec((1, _DIM, _NB), lambda n, k: (n, 0, 0)),
            pl.BlockSpec((_KB, _DIM), lambda n, k: (k, 0)),
        ],
        out_specs=[
            pl.BlockSpec((1, 1, _NB), lambda n, k: (n, 0, 0)),
            pl.BlockSpec(memory_space=pltpu.SMEM),
        ],
        out_shape=[
            jax.ShapeDtypeStruct((_B, 1, _HW), jnp.int32),
            jax.ShapeDtypeStruct((1, 1), jnp.float32),
        ],
        scratch_shapes=[
            pltpu.VMEM((_NB, _DIM), jnp.float32),
            pltpu.VMEM((_NB, 1), jnp.float32),
            pltpu.VMEM((_NB, 1), jnp.float32),
            pltpu.VMEM((_NB, 1), jnp.int32),
            pltpu.SMEM((1, 1), jnp.float32),
        ],
        compiler_params=pltpu.CompilerParams(
            dimension_semantics=("arbitrary", "arbitrary")),
    )(z_cn, emb)


def _sc_gather_kernel(idx_hbm, table_hbm, zq_hbm, util_hbm,
                      idx_v, rows_v, allidx_v, flags_v, util_v, sem):
    c = lax.axis_index("c")
    s = lax.axis_index("s")
    wid = s * 2 + c
    per_w = _N // 32
    base = wid * per_w
    pltpu.sync_copy(idx_hbm.at[pl.ds(base, per_w)], idx_v)
    pltpu.async_copy(table_hbm.at[idx_v], rows_v, sem).wait()
    pltpu.sync_copy(rows_v, zq_hbm.at[pl.ds(base, per_w)])

    @pl.when(jnp.logical_and(c == 0, s == 0))
    def _util():
        pltpu.sync_copy(idx_hbm, allidx_v)
        zeros16 = jnp.zeros((16,), jnp.float32)
        ones16 = jnp.ones((16,), jnp.float32)

        def zbody(i, carry):
            flags_v[pl.ds(i * 16, 16)] = zeros16
            return carry

        lax.fori_loop(0, _VOCAB // 16, zbody, 0)

        def sbody(i, carry):
            iv = allidx_v[pl.ds(i * 16, 16)]
            plsc.store_scatter(flags_v, [iv], ones16)
            return carry

        lax.fori_loop(0, _N // 16, sbody, 0)

        def rbody(i, acc):
            return acc + flags_v[pl.ds(i * 16, 16)]

        acc = lax.fori_loop(0, _VOCAB // 16, rbody, zeros16)
        tot = jnp.sum(acc, axis=0)
        util_v[...] = zeros16 + tot * (1.0 / _VOCAB)
        pltpu.sync_copy(util_v, util_hbm)


def _sc_gather(idx_flat, emb):
    mesh = plsc.VectorSubcoreMesh(core_axis_name="c", subcore_axis_name="s")
    per_w = _N // 32
    run = pl.kernel(
        _sc_gather_kernel,
        out_type=[
            jax.ShapeDtypeStruct((_N, _DIM), jnp.float32),
            jax.ShapeDtypeStruct((16,), jnp.float32),
        ],
        mesh=mesh,
        scratch_types=[
            pltpu.VMEM((per_w,), jnp.int32),
            pltpu.VMEM((per_w, _DIM), jnp.float32),
            pltpu.VMEM((_N,), jnp.int32),
            pltpu.VMEM((_VOCAB,), jnp.float32),
            pltpu.VMEM((16,), jnp.float32),
            pltpu.SemaphoreType.DMA,
        ],
        compiler_params=pltpu.CompilerParams(needs_layout_passes=False),
    )
    return run(idx_flat, emb)


def _transpose_kernel(rows_ref, out_ref):
    out_ref[0] = jnp.transpose(rows_ref[0], (1, 0))


def _to_bchw(zq_rows):
    return pl.pallas_call(
        _transpose_kernel,
        grid=(_B,),
        in_specs=[pl.BlockSpec((1, _HW, _DIM), lambda b: (b, 0, 0))],
        out_specs=pl.BlockSpec((1, _DIM, _HW), lambda b: (b, 0, 0)),
        out_shape=jax.ShapeDtypeStruct((_B, _DIM, _HW), jnp.float32),
        compiler_params=pltpu.CompilerParams(
            dimension_semantics=("arbitrary",)),
    )(zq_rows)


def kernel(z_e, embedding_weight):
    z_cn = z_e.reshape(_B, _DIM, _HW)
    idx, loss = _distance_argmin(z_cn, embedding_weight)
    idx_flat = idx.reshape(_N)
    zq_rows, util16 = _sc_gather(idx_flat, embedding_weight)
    zq = _to_bchw(zq_rows.reshape(_B, _HW, _DIM)).reshape(
        _B, _DIM, 32, 32)
    return zq, loss[0, 0], util16[0]


# speedup vs baseline: 1.0700x; 1.0700x over previous
import functools

import jax
import jax.numpy as jnp
from jax import lax
from jax.experimental import pallas as pl
from jax.experimental.pallas import tpu as pltpu
from jax.experimental.pallas import tpu_sc as plsc

_VOCAB = 8192
_DIM = 256
_BETA = 0.25
_B = 8
_HW = 1024
_N = _B * _HW
_NB = 1024
_KB = 1024
_NSTEP_K = _VOCAB // _KB


_BIG = 2 ** 30


def _argmin_kernel(z_ref, e_ref, y2_ref, x2_ref, idx_ref, loss_ref,
                   z2_s, x2_s, ds_s, macc_s, bacc_s, bv_s, loss_s):
    n = pl.program_id(0)
    k = pl.program_id(1)

    @pl.when(k == 0)
    def _prep():
        z_rows = jnp.transpose(z_ref[0], (1, 0))
        x2_s[...] = jnp.transpose(x2_ref[0], (1, 0))
        z2_s[...] = z_rows + z_rows
        macc_s[...] = jnp.full((_NB, 128), jnp.inf, jnp.float32)

    @pl.when(k < _NSTEP_K)
    def _phase_a():
        e = e_ref[...]
        mm2 = lax.dot_general(z2_s[...], e, (((1,), (1,)), ((), ())),
                              precision=lax.Precision.DEFAULT,
                              preferred_element_type=jnp.float32)
        d2 = (x2_s[...] + y2_ref[0]) - mm2
        dist = jnp.sqrt(jnp.maximum(d2, 0.0))
        ds_s[k] = dist
        m = macc_s[...]
        for j in range(_KB // 128):
            m = jnp.minimum(m, dist[:, j * 128:(j + 1) * 128])
        macc_s[...] = m

    @pl.when(k == _NSTEP_K - 1)
    def _rowmin():
        bv = jnp.min(macc_s[...], axis=1, keepdims=True)
        bv_s[...] = bv
        part = jnp.sum(bv * bv)

        @pl.when(n == 0)
        def _():
            loss_s[0, 0] = part

        @pl.when(n > 0)
        def _():
            loss_s[0, 0] = loss_s[0, 0] + part

    @pl.when(k >= _NSTEP_K)
    def _phase_b():
        k8 = k - _NSTEP_K
        dist = ds_s[k8]
        io = lax.broadcasted_iota(jnp.int32, (_NB, _KB), 1) + k8 * _KB
        cand = jnp.where(dist == bv_s[...], io, jnp.int32(_BIG))
        m = cand[:, 0:128]
        for j in range(1, _KB // 128):
            m = jnp.minimum(m, cand[:, j * 128:(j + 1) * 128])

        @pl.when(k == _NSTEP_K)
        def _():
            bacc_s[...] = m

        @pl.when(k > _NSTEP_K)
        def _():
            bacc_s[...] = jnp.minimum(bacc_s[...], m)

    @pl.when(k == 2 * _NSTEP_K - 1)
    def _fin():
        bi = jnp.min(bacc_s[...], axis=1, keepdims=True)
        idx_ref[0, 0, :] = jnp.reshape(jnp.transpose(bi, (1, 0)),
                                       (1, _NB))[0]

        @pl.when(n == _B - 1)
        def _():
            loss_ref[0, 0] = loss_s[0, 0] * (_BETA / (_N * _DIM))


def _distance_argmin(z_cn, emb, y2, x2):
    return pl.pallas_call(
        _argmin_kernel,
        grid=(_B, 2 * _NSTEP_K),
        in_specs=[
            pl.BlockSpec((1, _DIM, _NB), lambda n, k: (n, 0, 0)),
            pl.BlockSpec((_KB, _DIM),
                         lambda n, k: (jnp.minimum(k, _NSTEP_K - 1), 0)),
            pl.BlockSpec((1, 1, _KB),
                         lambda n, k: (jnp.minimum(k, _NSTEP_K - 1), 0, 0)),
            pl.BlockSpec((1, 1, _NB), lambda n, k: (n, 0, 0)),
        ],
        out_specs=[
            pl.BlockSpec((1, 1, _NB), lambda n, k: (n, 0, 0)),
            pl.BlockSpec(memory_space=pltpu.SMEM),
        ],
        out_shape=[
            jax.ShapeDtypeStruct((_B, 1, _HW), jnp.int32),
            jax.ShapeDtypeStruct((1, 1), jnp.float32),
        ],
        scratch_shapes=[
            pltpu.VMEM((_NB, _DIM), jnp.float32),
            pltpu.VMEM((_NB, 1), jnp.float32),
            pltpu.VMEM((_NSTEP_K, _NB, _KB), jnp.float32),
            pltpu.VMEM((_NB, 128), jnp.float32),
            pltpu.VMEM((_NB, 128), jnp.int32),
            pltpu.VMEM((_NB, 1), jnp.float32),
            pltpu.SMEM((1, 1), jnp.float32),
        ],
        compiler_params=pltpu.CompilerParams(
            dimension_semantics=("arbitrary", "arbitrary")),
    )(z_cn, emb, y2, x2)


def _sc_gather_kernel(idx_hbm, table_hbm, zq_hbm, util_hbm,
                      idx_v, rows_v, allidx_v, flags_v, util_v, sem):
    c = lax.axis_index("c")
    s = lax.axis_index("s")
    wid = s * 2 + c
    per_w = _N // 32
    base = wid * per_w
    pltpu.sync_copy(idx_hbm.at[pl.ds(base, per_w)], idx_v)
    pltpu.async_copy(table_hbm.at[idx_v], rows_v, sem).wait()
    pltpu.sync_copy(rows_v, zq_hbm.at[pl.ds(base, per_w)])

    @pl.when(jnp.logical_and(c == 0, s == 0))
    def _util():
        pltpu.sync_copy(idx_hbm, allidx_v)
        zeros16 = jnp.zeros((16,), jnp.float32)
        ones16 = jnp.ones((16,), jnp.float32)

        def zbody(i, carry):
            flags_v[pl.ds(i * 16, 16)] = zeros16
            return carry

        lax.fori_loop(0, _VOCAB // 16, zbody, 0)

        def sbody(i, carry):
            iv = allidx_v[pl.ds(i * 16, 16)]
            plsc.store_scatter(flags_v, [iv], ones16)
            return carry

        lax.fori_loop(0, _N // 16, sbody, 0)

        def rbody(i, acc):
            return acc + flags_v[pl.ds(i * 16, 16)]

        acc = lax.fori_loop(0, _VOCAB // 16, rbody, zeros16)
        tot = jnp.sum(acc, axis=0)
        util_v[...] = zeros16 + tot * (1.0 / _VOCAB)
        pltpu.sync_copy(util_v, util_hbm)


def _sc_gather(idx_flat, emb):
    mesh = plsc.VectorSubcoreMesh(core_axis_name="c", subcore_axis_name="s")
    per_w = _N // 32
    run = pl.kernel(
        _sc_gather_kernel,
        out_type=[
            jax.ShapeDtypeStruct((_N, _DIM), jnp.float32),
            jax.ShapeDtypeStruct((16,), jnp.float32),
        ],
        mesh=mesh,
        scratch_types=[
            pltpu.VMEM((per_w,), jnp.int32),
            pltpu.VMEM((per_w, _DIM), jnp.float32),
            pltpu.VMEM((_N,), jnp.int32),
            pltpu.VMEM((_VOCAB,), jnp.float32),
            pltpu.VMEM((16,), jnp.float32),
            pltpu.SemaphoreType.DMA,
        ],
        compiler_params=pltpu.CompilerParams(needs_layout_passes=False),
    )
    return run(idx_flat, emb)


def _transpose_kernel(rows_ref, out_ref):
    out_ref[0] = jnp.transpose(rows_ref[0], (1, 0))


def _to_bchw(zq_rows):
    return pl.pallas_call(
        _transpose_kernel,
        grid=(_B,),
        in_specs=[pl.BlockSpec((1, _HW, _DIM), lambda b: (b, 0, 0))],
        out_specs=pl.BlockSpec((1, _DIM, _HW), lambda b: (b, 0, 0)),
        out_shape=jax.ShapeDtypeStruct((_B, _DIM, _HW), jnp.float32),
        compiler_params=pltpu.CompilerParams(
            dimension_semantics=("arbitrary",)),
    )(zq_rows)


def kernel(z_e, embedding_weight):
    z_cn = z_e.reshape(_B, _DIM, _HW)
    y2 = jnp.sum(embedding_weight * embedding_weight, axis=1)
    z_t = jnp.transpose(z_e, (0, 2, 3, 1))
    x2 = jnp.sum(z_t * z_t, axis=3)
    idx, loss = _distance_argmin(z_cn, embedding_weight,
                                 y2.reshape(_NSTEP_K, 1, _KB),
                                 x2.reshape(_B, 1, _NB))
    idx_flat = idx.reshape(_N)
    zq_rows, util16 = _sc_gather(idx_flat, embedding_weight)
    zq = _to_bchw(zq_rows.reshape(_B, _HW, _DIM)).reshape(
        _B, _DIM, 32, 32)
    return zq, loss[0, 0], util16[0]
